# baseline (speedup 1.0000x reference)
import functools

import jax
import jax.numpy as jnp
from jax.experimental import pallas as pl
from jax.experimental.pallas import tpu as pltpu
from jax.experimental.pallas import tpu_sc as plsc

_NUM_LABELED = 100000
_HIDDEN = 64
_CLASSES = 100
_BATCH = 1024
_LABELED_WEIGHT = 0.7
_EPS = 1e-8

_K_TILE = 2000
_N_STEPS = _NUM_LABELED // _K_TILE

_GATHER_WINDOW = 128


def _simarg_body(wd_ref, m_ref, idx_ref, vmax_ref, imax_ref):
    step = pl.program_id(0)
    m = m_ref[...]
    ss = jnp.sum(m * m, axis=1, keepdims=True)
    inv = 1.0 / jnp.maximum(jnp.sqrt(ss), _EPS)
    mb = (m * inv).astype(jnp.bfloat16)
    wdb = wd_ref[...].astype(jnp.bfloat16)
    s = jax.lax.dot_general(
        wdb, mb, (((1,), (1,)), ((), ())),
        preferred_element_type=jnp.float32,
    )
    tmax = jnp.max(s, axis=1, keepdims=True)
    cols = jax.lax.broadcasted_iota(jnp.int32, s.shape, 1)
    targ = jnp.min(
        jnp.where(s == tmax, cols, jnp.int32(2**30)), axis=1, keepdims=True
    ) + step * _K_TILE

    @pl.when(step == 0)
    def _():
        vmax_ref[...] = tmax
        imax_ref[...] = targ

    @pl.when(step > 0)
    def _():
        upd = tmax > vmax_ref[...]
        vmax_ref[...] = jnp.where(upd, tmax, vmax_ref[...])
        imax_ref[...] = jnp.where(upd, targ, imax_ref[...])

    @pl.when(step == _N_STEPS - 1)
    def _():
        idx_ref[...] = imax_ref[...]


def _simarg(weak_data, labeled_memory):
    return pl.pallas_call(
        _simarg_body,
        grid=(_N_STEPS,),
        in_specs=[
            pl.BlockSpec((_BATCH, _HIDDEN), lambda i: (0, 0)),
            pl.BlockSpec((_K_TILE, _HIDDEN), lambda i: (i, 0)),
        ],
        out_specs=pl.BlockSpec((_BATCH, 1), lambda i: (0, 0)),
        out_shape=jax.ShapeDtypeStruct((_BATCH, 1), jnp.int32),
        scratch_shapes=[
            pltpu.VMEM((_BATCH, 1), jnp.float32),
            pltpu.VMEM((_BATCH, 1), jnp.int32),
        ],
    )(weak_data, labeled_memory)


_CPAD = 128


def _gather_sc(ids_2d, table):
    @functools.partial(
        pl.kernel,
        out_type=jax.ShapeDtypeStruct((_BATCH, _CPAD), jnp.float32),
        mesh=plsc.VectorSubcoreMesh(
            core_axis_name="core", subcore_axis_name="subcore"
        ),
    )
    def k(i_hbm, t_hbm, o_hbm):
        def body(i_vmem, o_vmem):
            pltpu.sync_copy(t_hbm.at[i_vmem.at[0]], o_vmem)

        pltpu.emit_pipeline(
            body,
            grid=(_BATCH // _GATHER_WINDOW,),
            in_specs=[
                pl.BlockSpec((1, _GATHER_WINDOW), index_map=lambda i: (0, i))
            ],
            out_specs=[
                pl.BlockSpec(
                    (_GATHER_WINDOW, _CPAD), index_map=lambda i: (i, 0)
                )
            ],
            core_axis_name=("core", "subcore"),
            dimension_semantics=(pltpu.PARALLEL,),
        )(i_hbm, o_hbm)

    return k(ids_2d, table)


def _blend_body(g_ref, w_ref, o_ref):
    o_ref[...] = g_ref[:, : _CLASSES] * _LABELED_WEIGHT + (
        1.0 - _NUM_LABELED
    ) * w_ref[...]


def _blend(g, weak_logits):
    return pl.pallas_call(
        _blend_body,
        out_shape=jax.ShapeDtypeStruct((_BATCH, _CLASSES), jnp.float32),
    )(g, weak_logits)


def kernel(weak_data, weak_logits, labeled_memory, labeled_logits):
    ids = _simarg(weak_data, labeled_memory)
    ids_2d = ids.reshape(1, _BATCH)
    table = jnp.pad(labeled_logits, ((0, 0), (0, _CPAD - _CLASSES)))
    g = _gather_sc(ids_2d, table)
    return _blend(g, weak_logits)
